# Initial kernel scaffold; baseline (speedup 1.0000x reference)
import jax
import jax.numpy as jnp
from jax.experimental import pallas as pl


def kernel(x, lut):
    raise NotImplementedError("write your pallas kernel here")


# speedup vs baseline: 1.5811x; 1.5811x over previous
import functools

import jax
import jax.numpy as jnp
from jax import lax
from jax.experimental import pallas as pl
from jax.experimental.pallas import tpu as pltpu
from jax.experimental.pallas import tpu_sc as plsc

VOCAB = 1000000
D = 32
BATCH = 16384
FIELDS = 26

NC = 2
NS = 16
NW = NC * NS

B_TOTAL = BATCH * FIELDS
B_PER_W = B_TOTAL // NW
CHUNK = 1024
NCHUNKS = B_PER_W // CHUNK
assert NCHUNKS * CHUNK == B_PER_W


def _emb_body(x_hbm, lut_hbm, out_hbm, idx_v, buf0, buf1, sem0, sem1):
    cid = lax.axis_index("c")
    sid = lax.axis_index("s")
    wid = sid * NC + cid
    pltpu.sync_copy(x_hbm.at[wid], idx_v)
    bufs = (buf0, buf1)
    sems = (sem0, sem1)

    def idx_slice(c):
        return idx_v.at[pl.ds(c * CHUNK, CHUNK)]

    copies = [None] * NCHUNKS
    copies[0] = pltpu.async_copy(lut_hbm.at[idx_slice(0)], bufs[0], sems[0])
    for c in range(NCHUNKS):
        if c + 1 < NCHUNKS:
            copies[c + 1] = pltpu.async_copy(
                lut_hbm.at[idx_slice(c + 1)], bufs[(c + 1) % 2], sems[(c + 1) % 2]
            )
        copies[c].wait()
        pltpu.sync_copy(bufs[c % 2], out_hbm.at[wid, c])


_emb = functools.partial(
    pl.kernel,
    out_type=jax.ShapeDtypeStruct((NW, NCHUNKS, CHUNK, D), jnp.float32),
    mesh=plsc.VectorSubcoreMesh(core_axis_name="c", subcore_axis_name="s"),
    scratch_types=[
        pltpu.VMEM((B_PER_W,), jnp.int32),
        pltpu.VMEM((CHUNK, D), jnp.float32),
        pltpu.VMEM((CHUNK, D), jnp.float32),
        pltpu.SemaphoreType.DMA,
        pltpu.SemaphoreType.DMA,
    ],
    compiler_params=pltpu.CompilerParams(use_tc_tiling_on_sc=False),
)(_emb_body)


@jax.jit
def kernel(x, lut):
    xi = x.reshape(NW, B_PER_W).astype(jnp.int32)
    out = _emb(xi, lut)
    return out.reshape(BATCH, FIELDS, D)
